# baseline (speedup 1.0000x reference)
import functools

import jax
import jax.numpy as jnp
from jax import lax
from jax.experimental import pallas as pl
from jax.experimental.pallas import tpu as pltpu
from jax.experimental.pallas import tpu_sc as plsc

R = 16384
C = 4096
G = 128
L = 16
NC = 2
NS = 16
NW = NC * NS
ROWS_PER_W = R // NW
KR = 8
CHUNKS = ROWS_PER_W // KR
QV = KR * G // L


def _body(x_hbm, inds_hbm, out_hbm, inds_v, idx_v, inbuf0, inbuf1,
          outbuf0, outbuf1, in_sem0, in_sem1, out_sem0, out_sem1):
    inbufs = (inbuf0, inbuf1)
    outbufs = (outbuf0, outbuf1)
    in_sems = (in_sem0, in_sem1)
    out_sems = (out_sem0, out_sem1)
    wid = lax.axis_index("s") * NC + lax.axis_index("c")
    row0 = wid * ROWS_PER_W

    pltpu.sync_copy(inds_hbm, inds_v)
    for j in range(KR):
        for t in range(G // L):
            idx_v[pl.ds(j * G + t * L, L)] = inds_v[pl.ds(t * L, L)] + j * C

    def fire_in(c, b):
        pltpu.async_copy(
            x_hbm.at[pl.ds((row0 + c * KR) * C, KR * C)],
            inbufs[b], in_sems[b],
        )

    def drain_in(b):
        pltpu.make_async_copy(
            x_hbm.at[pl.ds(0, KR * C)], inbufs[b], in_sems[b]
        ).wait()

    def subsample(b):
        for q in range(QV):
            g = plsc.load_gather(inbufs[b], [idx_v[pl.ds(q * L, L)]])
            outbufs[b][pl.ds(q * L, L)] = g

    def fire_out(c, b):
        pltpu.async_copy(
            outbufs[b],
            out_hbm.at[pl.ds((row0 + c * KR) * G, KR * G)], out_sems[b],
        )

    def drain_out(b):
        pltpu.make_async_copy(
            outbufs[b], out_hbm.at[pl.ds(0, KR * G)], out_sems[b]
        ).wait()

    fire_in(0, 0)
    fire_in(1, 1)

    for b in range(2):
        drain_in(b)
        subsample(b)
        fire_in(b + 2, b)
        fire_out(b, b)

    def group(gi, carry):
        for b in range(2):
            c = 2 * gi + b
            drain_in(b)
            drain_out(b)
            subsample(b)
            fire_in(c + 2, b)
            fire_out(c, b)
        return carry

    lax.fori_loop(1, CHUNKS // 2 - 1, group, 0)

    for b in range(2):
        c = CHUNKS - 2 + b
        drain_in(b)
        drain_out(b)
        subsample(b)
        fire_out(c, b)
    for b in range(2):
        drain_out(b)


@jax.jit
def kernel(x, inds):
    x_flat = x.reshape(R * C)
    inds_flat = inds.reshape(G).astype(jnp.int32)
    mesh = plsc.VectorSubcoreMesh(core_axis_name="c", subcore_axis_name="s")
    run = functools.partial(
        pl.kernel,
        mesh=mesh,
        compiler_params=pltpu.CompilerParams(needs_layout_passes=False),
        out_type=jax.ShapeDtypeStruct((R * G,), jnp.float32),
        scratch_types=[
            pltpu.VMEM((G,), jnp.int32),
            pltpu.VMEM((KR * G,), jnp.int32),
            pltpu.VMEM((KR * C,), jnp.float32),
            pltpu.VMEM((KR * C,), jnp.float32),
            pltpu.VMEM((KR * G,), jnp.float32),
            pltpu.VMEM((KR * G,), jnp.float32),
            pltpu.SemaphoreType.DMA,
            pltpu.SemaphoreType.DMA,
            pltpu.SemaphoreType.DMA,
            pltpu.SemaphoreType.DMA,
        ],
    )(_body)
    return run(x_flat, inds_flat).reshape(R, G)


# speedup vs baseline: 3.9037x; 3.9037x over previous
import jax
import jax.numpy as jnp
from jax import lax
from jax.experimental import pallas as pl
from jax.experimental.pallas import tpu as pltpu

R, C, G = 16384, 4096, 128
BR = 512


def _mm_body(inds_ref, x_ref, o_ref, sel_ref):
    @pl.when(pl.program_id(0) == 0)
    def _():
        iota = lax.broadcasted_iota(jnp.int32, (C, G), 0)
        sel_ref[...] = (iota == inds_ref[0, :][None, :]).astype(jnp.float32)

    o_ref[...] = jnp.dot(
        x_ref[...], sel_ref[...], preferred_element_type=jnp.float32
    )


def kernel(x, inds):
    inds32 = inds.astype(jnp.int32)
    return pl.pallas_call(
        _mm_body,
        grid=(R // BR,),
        in_specs=[
            pl.BlockSpec((1, G), lambda i: (0, 0)),
            pl.BlockSpec((BR, C), lambda i: (i, 0)),
        ],
        out_specs=pl.BlockSpec((BR, G), lambda i: (i, 0)),
        out_shape=jax.ShapeDtypeStruct((R, G), jnp.float32),
        scratch_shapes=[pltpu.VMEM((C, G), jnp.float32)],
    )(inds32, x)
